# Initial kernel scaffold; baseline (speedup 1.0000x reference)
import jax
import jax.numpy as jnp
from jax.experimental import pallas as pl


def kernel(x, edge_index, W, a1, a2):
    raise NotImplementedError("write your pallas kernel here")


# speedup vs baseline: 9.5452x; 9.5452x over previous
import functools

import jax
import jax.numpy as jnp
import numpy as np
from jax import lax
from jax.experimental import pallas as pl
from jax.experimental.pallas import tpu as pltpu
from jax.experimental.pallas import tpu_sc as plsc

NC = 2
NS = 16
LANES = 16
K = 128
ZR = 128


def _tc_gate_kernel(x_ref, w_ref, a2t_ref, y0_ref, y1_ref, r16_ref):
    sup = jnp.dot(x_ref[...], w_ref[...], preferred_element_type=jnp.float32)
    z = jnp.sum(sup * a2t_ref[...], axis=1, keepdims=True)
    z = jnp.where(z >= 0, z, 0.1 * z)
    r = jax.nn.sigmoid(z)
    y = sup * r
    h = y.shape[1] // 2
    y0_ref[...] = y[:, :h]
    y1_ref[...] = y[:, h:]
    r16_ref[...] = jnp.broadcast_to(r, (r.shape[0], LANES))


def _tc_final_kernel(a0_ref, a1_ref, s0_ref, s1_ref, o_ref):
    s = s0_ref[0, :, 0:1] + s1_ref[0, :, 0:1]
    v = jnp.concatenate([a0_ref[0], a1_ref[0]], axis=1) / s
    o_ref[...] = jnp.where(v > 0, v, jnp.exp(jnp.minimum(v, 0.0)) - 1.0)


def _make_sc_spmm(n_pad, h, c_per_tile):
    seg = n_pad // NS
    mesh = plsc.VectorSubcoreMesh(
        core_axis_name="c", subcore_axis_name="s",
        num_cores=NC, num_subcores=NS)
    f32 = jnp.float32

    @functools.partial(
        pl.kernel,
        out_type=[
            jax.ShapeDtypeStruct((NC, n_pad, h), f32),
            jax.ShapeDtypeStruct((NC, n_pad, LANES), f32),
            jax.ShapeDtypeStruct((NC, n_pad, h), f32),
            jax.ShapeDtypeStruct((NC, n_pad, h), f32),
        ],
        mesh=mesh,
        scratch_types=[
            pltpu.VMEM_SHARED((n_pad, h), f32),
            pltpu.VMEM_SHARED((n_pad, LANES), f32),
            pltpu.VMEM((c_per_tile, K), jnp.int32),
            pltpu.VMEM((c_per_tile, K), jnp.int32),
            pltpu.VMEM((2, K, h), f32),
            pltpu.VMEM((2, K, LANES), f32),
            pltpu.VMEM((ZR, h), f32),
            pltpu.VMEM((ZR, LANES), f32),
            pltpu.SemaphoreType.DMA,
            pltpu.SemaphoreType.DMA,
        ],
        compiler_params=pltpu.CompilerParams(use_tc_tiling_on_sc=False),
    )
    def sc_spmm(y2_hbm, r16_hbm, colc_hbm, rowc_hbm,
                a2_hbm, s2_hbm, t1_hbm, t2_hbm,
                Q, S16, colv, rowv, gbuf, g16, zb, zb16,
                semg, sems):
        c = lax.axis_index("c")
        s = lax.axis_index("s")
        base = s * seg

        def stream_phase(src, dst, bufs, lo, n_chunks):
            def fire_g(j, bank):
                pltpu.async_copy(src.at[colv.at[lo + j]], bufs.at[bank],
                                 semg)

            def drain_g(j, bank):
                pltpu.make_async_copy(src.at[colv.at[lo + j]],
                                      bufs.at[bank], semg).wait()

            def fire_s(j, bank):
                pltpu.async_copy(bufs.at[bank], dst.at[rowv.at[lo + j]],
                                 sems, add=True)

            def drain_s(j, bank):
                pltpu.make_async_copy(bufs.at[bank],
                                      dst.at[rowv.at[lo + j]], sems).wait()

            fire_g(0, 0)
            drain_g(0, 0)
            fire_s(0, 0)

            def body(g, carry):
                bank = g % 2
                fire_g(g, bank)
                drain_s(g - 1, 1 - bank)
                drain_g(g, bank)
                fire_s(g, bank)
                return carry

            lax.fori_loop(1, n_chunks, body, 0)
            last = n_chunks - 1
            drain_s(last, last % 2)

        def zero_rows(dst, zsrc):
            nfull = seg // ZR
            rem = seg - nfull * ZR
            for t in range(nfull):
                pltpu.sync_copy(zsrc, dst.at[pl.ds(base + t * ZR, ZR)])
            if rem:
                pltpu.sync_copy(zsrc.at[pl.ds(0, rem)],
                                dst.at[pl.ds(base + nfull * ZR, rem)])

        def flush_rows(dst2):
            pltpu.sync_copy(Q.at[pl.ds(base, seg)],
                            dst2.at[c].at[pl.ds(base, seg)])

        zv = jnp.zeros((LANES,), f32)
        for i in range(ZR):
            for q in range(h // LANES):
                zb[i, pl.ds(q * LANES, LANES)] = zv
            zb16[i, pl.ds(0, LANES)] = zv
        pltpu.sync_copy(colc_hbm.at[pl.ds(s * c_per_tile, c_per_tile)], colv)
        pltpu.sync_copy(rowc_hbm.at[pl.ds(s * c_per_tile, c_per_tile)], rowv)
        zero_rows(Q, zb)
        zero_rows(S16, zb16)
        plsc.subcore_barrier()

        half_c = c_per_tile // 2
        stream_phase(r16_hbm, S16, g16, c * half_c, half_c)
        stream_phase(y2_hbm.at[c], Q, gbuf, 0, c_per_tile)
        plsc.subcore_barrier()

        flush_rows(t1_hbm)
        pltpu.sync_copy(S16.at[pl.ds(base, seg)],
                        s2_hbm.at[c].at[pl.ds(base, seg)])
        zero_rows(Q, zb)
        plsc.subcore_barrier()

        stream_phase(t1_hbm.at[c], Q, gbuf, 0, c_per_tile)
        plsc.subcore_barrier()
        flush_rows(t2_hbm)
        zero_rows(Q, zb)
        plsc.subcore_barrier()

        stream_phase(t2_hbm.at[c], Q, gbuf, 0, c_per_tile)
        plsc.subcore_barrier()
        flush_rows(a2_hbm)

    return sc_spmm


def kernel(x, edge_index, W, a1, a2):
    del a1
    n, d_in = x.shape
    d_out = W.shape[1]
    h = d_out // 2
    e = edge_index.shape[1]
    f32 = jnp.float32

    n_pad = -(-n // (NS * 8)) * (NS * 8)
    if n_pad == n:
        n_pad += NS * 8
    ept = -(-e // NS)
    ept = -(-ept // (8 * K)) * (8 * K)
    c_per_tile = ept // K
    e_pad = ept * NS - e

    row = edge_index[0]
    col = edge_index[1]
    if e_pad:
        pr = n + (np.arange(e_pad, dtype=np.int32) % (n_pad - n))
        pc = (np.arange(e_pad, dtype=np.int32) * 131) % n
        row = jnp.concatenate([row, jnp.asarray(pr)])
        col = jnp.concatenate([col, jnp.asarray(pc)])
    rowc = row.reshape(NS * c_per_tile, K)
    colc = col.reshape(NS * c_per_tile, K)

    bn = 1000
    grid = n // bn
    y0, y1, r16 = pl.pallas_call(
        _tc_gate_kernel,
        grid=(grid,),
        in_specs=[
            pl.BlockSpec((bn, d_in), lambda i: (i, 0)),
            pl.BlockSpec((d_in, d_out), lambda i: (0, 0)),
            pl.BlockSpec((1, d_out), lambda i: (0, 0)),
        ],
        out_specs=[
            pl.BlockSpec((bn, h), lambda i: (i, 0)),
            pl.BlockSpec((bn, h), lambda i: (i, 0)),
            pl.BlockSpec((bn, LANES), lambda i: (i, 0)),
        ],
        out_shape=[
            jax.ShapeDtypeStruct((n_pad, h), f32),
            jax.ShapeDtypeStruct((n_pad, h), f32),
            jax.ShapeDtypeStruct((n_pad, LANES), f32),
        ],
    )(x, W, a2.reshape(1, d_out))
    y2 = jnp.stack([y0, y1])

    sc_spmm = _make_sc_spmm(n_pad, h, c_per_tile)
    a2h, s2, _, _ = sc_spmm(y2, r16, colc, rowc)

    out = pl.pallas_call(
        _tc_final_kernel,
        grid=(grid,),
        in_specs=[
            pl.BlockSpec((1, bn, h), lambda i: (0, i, 0)),
            pl.BlockSpec((1, bn, h), lambda i: (1, i, 0)),
            pl.BlockSpec((1, bn, LANES), lambda i: (0, i, 0)),
            pl.BlockSpec((1, bn, LANES), lambda i: (1, i, 0)),
        ],
        out_specs=pl.BlockSpec((bn, d_out), lambda i: (i, 0)),
        out_shape=jax.ShapeDtypeStruct((n, d_out), f32),
    )(a2h, a2h, s2, s2)
    return out


# speedup vs baseline: 14.4994x; 1.5190x over previous
import functools

import jax
import jax.numpy as jnp
import numpy as np
from jax import lax
from jax.experimental import pallas as pl
from jax.experimental.pallas import tpu as pltpu
from jax.experimental.pallas import tpu_sc as plsc

NC = 2
NS = 16
LANES = 16
K = 128
ZR = 64


def _tc_gate_kernel(x_ref, w_ref, a2t_ref, y0_ref, y1_ref, r16_ref):
    sup = jnp.dot(x_ref[...], w_ref[...], preferred_element_type=jnp.float32)
    z = jnp.sum(sup * a2t_ref[...], axis=1, keepdims=True)
    z = jnp.where(z >= 0, z, 0.1 * z)
    r = jax.nn.sigmoid(z)
    y = sup * r
    h = y.shape[1] // 2
    y0_ref[...] = y[:, :h]
    y1_ref[...] = y[:, h:]
    r16_ref[...] = jnp.broadcast_to(r, (r.shape[0], LANES))


def _tc_final_kernel(a0_ref, a1_ref, s0_ref, s1_ref, o_ref):
    s = s0_ref[0, :, 0:1] + s1_ref[0, :, 0:1]
    v = jnp.concatenate([a0_ref[0], a1_ref[0]], axis=1) / s
    o_ref[...] = jnp.where(v > 0, v, jnp.exp(jnp.minimum(v, 0.0)) - 1.0)


def _make_sc_spmm(n_pad, h, c_per_tile):
    seg = n_pad // NS
    mesh = plsc.VectorSubcoreMesh(
        core_axis_name="c", subcore_axis_name="s",
        num_cores=NC, num_subcores=NS)
    f32 = jnp.float32

    @functools.partial(
        pl.kernel,
        out_type=[
            jax.ShapeDtypeStruct((NC, n_pad, h), f32),
            jax.ShapeDtypeStruct((NC, n_pad, LANES), f32),
            jax.ShapeDtypeStruct((NC, n_pad, h), f32),
            jax.ShapeDtypeStruct((NC, n_pad, h), f32),
        ],
        mesh=mesh,
        scratch_types=[
            pltpu.VMEM_SHARED((n_pad, h), f32),
            pltpu.VMEM_SHARED((n_pad, LANES), f32),
            pltpu.VMEM((c_per_tile, K), jnp.int32),
            pltpu.VMEM((c_per_tile, K), jnp.int32),
            pltpu.VMEM((3, K, h), f32),
            pltpu.VMEM((2, K, LANES), f32),
            pltpu.VMEM((ZR, h), f32),
            pltpu.VMEM((ZR, LANES), f32),
            pltpu.SemaphoreType.DMA,
            pltpu.SemaphoreType.DMA,
        ],
        compiler_params=pltpu.CompilerParams(use_tc_tiling_on_sc=False),
    )
    def sc_spmm(y2_hbm, r16_hbm, colc_hbm, rowc_hbm,
                a2_hbm, s2_hbm, t1_hbm, t2_hbm,
                Q, S16, colv, rowv, gbuf, g16, zb, zb16,
                semg, sems):
        c = lax.axis_index("c")
        s = lax.axis_index("s")
        base = s * seg

        def stream_phase(src, dst, bufs, depth, sd, lo, n_chunks):
            def fire_g(j):
                pltpu.async_copy(src.at[colv.at[lo + j]],
                                 bufs.at[j % depth], semg)

            def drain_g(j):
                pltpu.make_async_copy(src.at[colv.at[lo + j]],
                                      bufs.at[j % depth], semg).wait()

            def fire_s(j):
                pltpu.async_copy(bufs.at[j % depth],
                                 dst.at[rowv.at[lo + j]], sems, add=True)

            def drain_s(j):
                pltpu.make_async_copy(bufs.at[j % depth],
                                      dst.at[rowv.at[lo + j]], sems).wait()

            for j in range(depth - sd):
                fire_g(j)

            def body(g, carry):
                @pl.when(g >= sd)
                def _():
                    drain_s(g - sd)

                @pl.when(g + (depth - sd) < n_chunks)
                def _():
                    fire_g(g + (depth - sd))

                drain_g(g)
                fire_s(g)
                return carry

            lax.fori_loop(0, n_chunks, body, 0)
            for j in range(n_chunks - sd, n_chunks):
                drain_s(j)

        def zero_rows(dst, zsrc):
            nfull = seg // ZR
            rem = seg - nfull * ZR
            for t in range(nfull):
                pltpu.sync_copy(zsrc, dst.at[pl.ds(base + t * ZR, ZR)])
            if rem:
                pltpu.sync_copy(zsrc.at[pl.ds(0, rem)],
                                dst.at[pl.ds(base + nfull * ZR, rem)])

        def flush_rows(dst2):
            pltpu.sync_copy(Q.at[pl.ds(base, seg)],
                            dst2.at[c].at[pl.ds(base, seg)])

        zv = jnp.zeros((LANES,), f32)
        for i in range(ZR):
            for q in range(h // LANES):
                zb[i, pl.ds(q * LANES, LANES)] = zv
            zb16[i, pl.ds(0, LANES)] = zv
        pltpu.sync_copy(colc_hbm.at[pl.ds(s * c_per_tile, c_per_tile)], colv)
        pltpu.sync_copy(rowc_hbm.at[pl.ds(s * c_per_tile, c_per_tile)], rowv)
        zero_rows(Q, zb)
        zero_rows(S16, zb16)
        plsc.subcore_barrier()

        half_c = c_per_tile // 2
        stream_phase(r16_hbm, S16, g16, 2, 1, c * half_c, half_c)
        stream_phase(y2_hbm.at[c], Q, gbuf, 3, 2, 0, c_per_tile)
        plsc.subcore_barrier()

        flush_rows(t1_hbm)
        pltpu.sync_copy(S16.at[pl.ds(base, seg)],
                        s2_hbm.at[c].at[pl.ds(base, seg)])
        zero_rows(Q, zb)
        plsc.subcore_barrier()

        stream_phase(t1_hbm.at[c], Q, gbuf, 3, 2, 0, c_per_tile)
        plsc.subcore_barrier()
        flush_rows(t2_hbm)
        zero_rows(Q, zb)
        plsc.subcore_barrier()

        stream_phase(t2_hbm.at[c], Q, gbuf, 3, 2, 0, c_per_tile)
        plsc.subcore_barrier()
        flush_rows(a2_hbm)

    return sc_spmm


def kernel(x, edge_index, W, a1, a2):
    del a1
    n, d_in = x.shape
    d_out = W.shape[1]
    h = d_out // 2
    e = edge_index.shape[1]
    f32 = jnp.float32

    n_pad = -(-n // (NS * 8)) * (NS * 8)
    if n_pad == n:
        n_pad += NS * 8
    ept = -(-e // NS)
    ept = -(-ept // (8 * K)) * (8 * K)
    c_per_tile = ept // K
    e_pad = ept * NS - e

    row = edge_index[0]
    col = edge_index[1]
    if e_pad:
        pr = n + (np.arange(e_pad, dtype=np.int32) % (n_pad - n))
        pc = (np.arange(e_pad, dtype=np.int32) * 131) % n
        row = jnp.concatenate([row, jnp.asarray(pr)])
        col = jnp.concatenate([col, jnp.asarray(pc)])
    rowc = row.reshape(NS * c_per_tile, K)
    colc = col.reshape(NS * c_per_tile, K)

    bn = 1000
    grid = n // bn
    y0, y1, r16 = pl.pallas_call(
        _tc_gate_kernel,
        grid=(grid,),
        in_specs=[
            pl.BlockSpec((bn, d_in), lambda i: (i, 0)),
            pl.BlockSpec((d_in, d_out), lambda i: (0, 0)),
            pl.BlockSpec((1, d_out), lambda i: (0, 0)),
        ],
        out_specs=[
            pl.BlockSpec((bn, h), lambda i: (i, 0)),
            pl.BlockSpec((bn, h), lambda i: (i, 0)),
            pl.BlockSpec((bn, LANES), lambda i: (i, 0)),
        ],
        out_shape=[
            jax.ShapeDtypeStruct((n_pad, h), f32),
            jax.ShapeDtypeStruct((n_pad, h), f32),
            jax.ShapeDtypeStruct((n_pad, LANES), f32),
        ],
    )(x, W, a2.reshape(1, d_out))
    y2 = jnp.stack([y0, y1])

    sc_spmm = _make_sc_spmm(n_pad, h, c_per_tile)
    a2h, s2, _, _ = sc_spmm(y2, r16, colc, rowc)

    out = pl.pallas_call(
        _tc_final_kernel,
        grid=(grid,),
        in_specs=[
            pl.BlockSpec((1, bn, h), lambda i: (0, i, 0)),
            pl.BlockSpec((1, bn, h), lambda i: (1, i, 0)),
            pl.BlockSpec((1, bn, LANES), lambda i: (0, i, 0)),
            pl.BlockSpec((1, bn, LANES), lambda i: (1, i, 0)),
        ],
        out_specs=pl.BlockSpec((bn, d_out), lambda i: (i, 0)),
        out_shape=jax.ShapeDtypeStruct((n, d_out), f32),
    )(a2h, a2h, s2, s2)
    return out


# speedup vs baseline: 14.8800x; 1.0262x over previous
import functools

import jax
import jax.numpy as jnp
import numpy as np
from jax import lax
from jax.experimental import pallas as pl
from jax.experimental.pallas import tpu as pltpu
from jax.experimental.pallas import tpu_sc as plsc

NC = 2
NS = 16
LANES = 16
K = 128
ZR = 64


def _tc_gate_kernel(x_ref, w_ref, a2t_ref, y0_ref, y1_ref, r16_ref):
    sup = jnp.dot(x_ref[...], w_ref[...], preferred_element_type=jnp.float32)
    z = jnp.sum(sup * a2t_ref[...], axis=1, keepdims=True)
    z = jnp.where(z >= 0, z, 0.1 * z)
    r = jax.nn.sigmoid(z)
    y = sup * r
    h = y.shape[1] // 2
    y0_ref[...] = y[:, :h]
    y1_ref[...] = y[:, h:]
    r16_ref[...] = jnp.broadcast_to(r, (r.shape[0], LANES))


def _tc_final_kernel(a0_ref, a1_ref, s0_ref, s1_ref, o_ref):
    s = s0_ref[0, :, 0:1] + s1_ref[0, :, 0:1]
    v = jnp.concatenate([a0_ref[0], a1_ref[0]], axis=1) / s
    o_ref[...] = jnp.where(v > 0, v, jnp.exp(jnp.minimum(v, 0.0)) - 1.0)


def _make_sc_spmm(n_pad, h, c_per_tile, e_total):
    seg = n_pad // NS
    mesh = plsc.VectorSubcoreMesh(
        core_axis_name="c", subcore_axis_name="s",
        num_cores=NC, num_subcores=NS)
    f32 = jnp.float32

    @functools.partial(
        pl.kernel,
        out_type=[
            jax.ShapeDtypeStruct((NC, n_pad, h), f32),
            jax.ShapeDtypeStruct((NC, n_pad, LANES), f32),
            jax.ShapeDtypeStruct((NC, n_pad, h), f32),
            jax.ShapeDtypeStruct((NC, n_pad, h), f32),
        ],
        mesh=mesh,
        scratch_types=[
            pltpu.VMEM_SHARED((n_pad, h), f32),
            pltpu.VMEM_SHARED((n_pad, LANES), f32),
            pltpu.VMEM((c_per_tile * K,), jnp.int32),
            pltpu.VMEM((c_per_tile, K), jnp.int32),
            pltpu.VMEM((3, K, h), f32),
            pltpu.VMEM((2, K, LANES), f32),
            pltpu.VMEM((ZR, h), f32),
            pltpu.VMEM((ZR, LANES), f32),
            pltpu.SemaphoreType.DMA,
            pltpu.SemaphoreType.DMA,
        ],
        compiler_params=pltpu.CompilerParams(use_tc_tiling_on_sc=False),
    )
    def sc_spmm(y0_hbm, y1_hbm, r16_hbm, ei_hbm, pcol_hbm, prow_hbm,
                a2_hbm, s2_hbm, t1_hbm, t2_hbm,
                Q, S16, colv, rowv, gbuf, g16, zb, zb16,
                semg, sems):
        c = lax.axis_index("c")
        s = lax.axis_index("s")
        base = s * seg

        def stream_phase(src, dst, bufs, depth, sd, lo, n_chunks):
            def fire_g(j):
                pltpu.async_copy(src.at[colv.at[pl.ds((lo + j) * K, K)]],
                                 bufs.at[j % depth], semg)

            def drain_g(j):
                pltpu.make_async_copy(src.at[colv.at[pl.ds((lo + j) * K, K)]],
                                      bufs.at[j % depth], semg).wait()

            def fire_s(j):
                pltpu.async_copy(bufs.at[j % depth],
                                 dst.at[rowv.at[lo + j]], sems, add=True)

            def drain_s(j):
                pltpu.make_async_copy(bufs.at[j % depth],
                                      dst.at[rowv.at[lo + j]], sems).wait()

            for j in range(depth - sd):
                fire_g(j)

            def body(g, carry):
                @pl.when(g >= sd)
                def _():
                    drain_s(g - sd)

                @pl.when(g + (depth - sd) < n_chunks)
                def _():
                    fire_g(g + (depth - sd))

                drain_g(g)
                fire_s(g)
                return carry

            lax.fori_loop(0, n_chunks, body, 0)
            for j in range(n_chunks - sd, n_chunks):
                drain_s(j)

        def zero_rows(dst, zsrc):
            nfull = seg // ZR
            rem = seg - nfull * ZR
            for t in range(nfull):
                pltpu.sync_copy(zsrc, dst.at[pl.ds(base + t * ZR, ZR)])
            if rem:
                pltpu.sync_copy(zsrc.at[pl.ds(0, rem)],
                                dst.at[pl.ds(base + nfull * ZR, rem)])

        def flush_rows(dst2):
            pltpu.sync_copy(Q.at[pl.ds(base, seg)],
                            dst2.at[c].at[pl.ds(base, seg)])

        zv = jnp.zeros((LANES,), f32)
        for i in range(ZR):
            for q in range(h // LANES):
                zb[i, pl.ds(q * LANES, LANES)] = zv
            zb16[i, pl.ds(0, LANES)] = zv
        ept = c_per_tile * K
        ecol = ei_hbm.at[0]
        erow = ei_hbm.at[1]

        @pl.when(s < NS - 1)
        def _():
            pltpu.sync_copy(ecol.at[pl.ds(s * ept, ept)], colv)

        @pl.when(s == NS - 1)
        def _():
            rem = e_total - (NS - 1) * ept
            pltpu.sync_copy(ecol.at[pl.ds((NS - 1) * ept, rem)],
                            colv.at[pl.ds(0, rem)])
            pltpu.sync_copy(pcol_hbm.at[pl.ds(0, ept - rem)],
                            colv.at[pl.ds(rem, ept - rem)])

        main_chunks = e_total // K

        def stage_rows(g, carry):
            gg = s * c_per_tile + g

            @pl.when(gg < main_chunks)
            def _():
                pltpu.async_copy(erow.at[pl.ds(gg * K, K)], rowv.at[g], semg)

            @pl.when(gg >= main_chunks)
            def _():
                pltpu.async_copy(prow_hbm.at[gg - main_chunks], rowv.at[g],
                                 semg)

            @pl.when(g >= 8)
            def _():
                pltpu.make_async_copy(prow_hbm.at[0], rowv.at[g - 8],
                                      semg).wait()
            return carry

        lax.fori_loop(0, c_per_tile, stage_rows, 0)
        for j in range(c_per_tile - 8, c_per_tile):
            pltpu.make_async_copy(prow_hbm.at[0], rowv.at[j], semg).wait()
        zero_rows(Q, zb)
        zero_rows(S16, zb16)
        plsc.subcore_barrier()

        half_c = c_per_tile // 2
        stream_phase(r16_hbm, S16, g16, 2, 1, c * half_c, half_c)

        @pl.when(c == 0)
        def _():
            stream_phase(y0_hbm, Q, gbuf, 3, 2, 0, c_per_tile)

        @pl.when(c != 0)
        def _():
            stream_phase(y1_hbm, Q, gbuf, 3, 2, 0, c_per_tile)
        plsc.subcore_barrier()

        flush_rows(t1_hbm)
        pltpu.sync_copy(S16.at[pl.ds(base, seg)],
                        s2_hbm.at[c].at[pl.ds(base, seg)])
        zero_rows(Q, zb)
        plsc.subcore_barrier()

        stream_phase(t1_hbm.at[c], Q, gbuf, 3, 2, 0, c_per_tile)
        plsc.subcore_barrier()
        flush_rows(t2_hbm)
        zero_rows(Q, zb)
        plsc.subcore_barrier()

        stream_phase(t2_hbm.at[c], Q, gbuf, 3, 2, 0, c_per_tile)
        plsc.subcore_barrier()
        flush_rows(a2_hbm)

    return sc_spmm


def kernel(x, edge_index, W, a1, a2):
    del a1
    n, d_in = x.shape
    d_out = W.shape[1]
    h = d_out // 2
    e = edge_index.shape[1]
    f32 = jnp.float32

    n_pad = -(-n // (NS * 8)) * (NS * 8)
    if n_pad == n:
        n_pad += NS * 8
    ept = -(-e // NS)
    ept = -(-ept // (8 * K)) * (8 * K)
    c_per_tile = ept // K
    e_pad = ept * NS - e

    pr = (n + (np.arange(e_pad, dtype=np.int32) % (n_pad - n))).reshape(
        e_pad // K, K)
    pc = (np.arange(e_pad, dtype=np.int32) * 131) % n
    pad_row = jnp.asarray(pr, dtype=jnp.int32)
    pad_col = jnp.asarray(pc, dtype=jnp.int32)

    bn = 1000
    grid = n // bn
    y0, y1, r16 = pl.pallas_call(
        _tc_gate_kernel,
        grid=(grid,),
        in_specs=[
            pl.BlockSpec((bn, d_in), lambda i: (i, 0)),
            pl.BlockSpec((d_in, d_out), lambda i: (0, 0)),
            pl.BlockSpec((1, d_out), lambda i: (0, 0)),
        ],
        out_specs=[
            pl.BlockSpec((bn, h), lambda i: (i, 0)),
            pl.BlockSpec((bn, h), lambda i: (i, 0)),
            pl.BlockSpec((bn, LANES), lambda i: (i, 0)),
        ],
        out_shape=[
            jax.ShapeDtypeStruct((n_pad, h), f32),
            jax.ShapeDtypeStruct((n_pad, h), f32),
            jax.ShapeDtypeStruct((n_pad, LANES), f32),
        ],
    )(x, W, a2.reshape(1, d_out))
    sc_spmm = _make_sc_spmm(n_pad, h, c_per_tile, e)
    a2h, s2, _, _ = sc_spmm(y0, y1, r16, edge_index, pad_col, pad_row)

    out = pl.pallas_call(
        _tc_final_kernel,
        grid=(grid,),
        in_specs=[
            pl.BlockSpec((1, bn, h), lambda i: (0, i, 0)),
            pl.BlockSpec((1, bn, h), lambda i: (1, i, 0)),
            pl.BlockSpec((1, bn, LANES), lambda i: (0, i, 0)),
            pl.BlockSpec((1, bn, LANES), lambda i: (1, i, 0)),
        ],
        out_specs=pl.BlockSpec((bn, d_out), lambda i: (i, 0)),
        out_shape=jax.ShapeDtypeStruct((n, d_out), f32),
    )(a2h, a2h, s2, s2)
    return out


# speedup vs baseline: 14.9170x; 1.0025x over previous
import functools

import jax
import jax.numpy as jnp
import numpy as np
from jax import lax
from jax.experimental import pallas as pl
from jax.experimental.pallas import tpu as pltpu
from jax.experimental.pallas import tpu_sc as plsc

NC = 2
NS = 16
LANES = 16
K = 128
ZR = 64


def _tc_gate_kernel(x_ref, w_ref, a2t_ref, y0_ref, y1_ref, r16_ref):
    sup = jnp.dot(x_ref[...], w_ref[...], preferred_element_type=jnp.float32)
    z = jnp.sum(sup * a2t_ref[...], axis=1, keepdims=True)
    z = jnp.where(z >= 0, z, 0.1 * z)
    r = jax.nn.sigmoid(z)
    y = sup * r
    h = y.shape[1] // 2
    y0_ref[...] = y[:, :h]
    y1_ref[...] = y[:, h:]
    r16_ref[...] = jnp.broadcast_to(r, (r.shape[0], LANES))


def _tc_final_kernel(a0_ref, a1_ref, s0_ref, s1_ref, o_ref):
    s = s0_ref[0, :, 0:1] + s1_ref[0, :, 0:1]
    v = jnp.concatenate([a0_ref[0], a1_ref[0]], axis=1) / s
    o_ref[...] = jnp.where(v > 0, v, jnp.exp(jnp.minimum(v, 0.0)) - 1.0)


def _make_sc_spmm(n_pad, h, c_per_tile, e_total):
    seg = n_pad // NS
    mesh = plsc.VectorSubcoreMesh(
        core_axis_name="c", subcore_axis_name="s",
        num_cores=NC, num_subcores=NS)
    f32 = jnp.float32

    @functools.partial(
        pl.kernel,
        out_type=[
            jax.ShapeDtypeStruct((NC, n_pad, h), f32),
            jax.ShapeDtypeStruct((NC, n_pad, LANES), f32),
            jax.ShapeDtypeStruct((NC, n_pad, h), f32),
            jax.ShapeDtypeStruct((NC, n_pad, h), f32),
        ],
        mesh=mesh,
        scratch_types=[
            pltpu.VMEM_SHARED((n_pad, h), f32),
            pltpu.VMEM_SHARED((n_pad, LANES), f32),
            pltpu.VMEM((c_per_tile, K), jnp.int32),
            pltpu.VMEM((c_per_tile, K), jnp.int32),
            pltpu.VMEM((3, K, h), f32),
            pltpu.VMEM((2, K, LANES), f32),
            pltpu.VMEM((ZR, h), f32),
            pltpu.VMEM((ZR, LANES), f32),
            pltpu.SemaphoreType.DMA,
            pltpu.SemaphoreType.DMA,
        ],
        compiler_params=pltpu.CompilerParams(use_tc_tiling_on_sc=False),
    )
    def sc_spmm(y0_hbm, y1_hbm, r16_hbm, ei_hbm, pcol_hbm, prow_hbm,
                a2_hbm, s2_hbm, t1_hbm, t2_hbm,
                Q, S16, colv, rowv, gbuf, g16, zb, zb16,
                semg, sems):
        c = lax.axis_index("c")
        s = lax.axis_index("s")
        base = s * seg

        def stream_phase(src, dst, bufs, depth, sd, lo, n_chunks):
            def fire_g(j):
                pltpu.async_copy(src.at[colv.at[lo + j]],
                                 bufs.at[j % depth], semg)

            def drain_g(j):
                pltpu.make_async_copy(src.at[colv.at[lo + j]],
                                      bufs.at[j % depth], semg).wait()

            def fire_s(j):
                pltpu.async_copy(bufs.at[j % depth],
                                 dst.at[rowv.at[lo + j]], sems, add=True)

            def drain_s(j):
                pltpu.make_async_copy(bufs.at[j % depth],
                                      dst.at[rowv.at[lo + j]], sems).wait()

            for j in range(depth - sd):
                fire_g(j)

            def body(g, carry):
                @pl.when(g >= sd)
                def _():
                    drain_s(g - sd)

                @pl.when(g + (depth - sd) < n_chunks)
                def _():
                    fire_g(g + (depth - sd))

                drain_g(g)
                fire_s(g)
                return carry

            lax.fori_loop(0, n_chunks, body, 0)
            for j in range(n_chunks - sd, n_chunks):
                drain_s(j)

        def zero_rows(dst, zsrc):
            nfull = seg // ZR
            rem = seg - nfull * ZR
            for t in range(nfull):
                pltpu.sync_copy(zsrc, dst.at[pl.ds(base + t * ZR, ZR)])
            if rem:
                pltpu.sync_copy(zsrc.at[pl.ds(0, rem)],
                                dst.at[pl.ds(base + nfull * ZR, rem)])

        def flush_rows(dst2):
            pltpu.sync_copy(Q.at[pl.ds(base, seg)],
                            dst2.at[c].at[pl.ds(base, seg)])

        zv = jnp.zeros((LANES,), f32)
        for i in range(ZR):
            for q in range(h // LANES):
                zb[i, pl.ds(q * LANES, LANES)] = zv
            zb16[i, pl.ds(0, LANES)] = zv
        ecol = ei_hbm.at[0]
        erow = ei_hbm.at[1]
        main_chunks = e_total // K

        def stage_idx(g, carry):
            gg = s * c_per_tile + g

            @pl.when(gg < main_chunks)
            def _():
                pltpu.async_copy(ecol.at[pl.ds(gg * K, K)], colv.at[g], semg)
                pltpu.async_copy(erow.at[pl.ds(gg * K, K)], rowv.at[g], semg)

            @pl.when(gg >= main_chunks)
            def _():
                pltpu.async_copy(pcol_hbm.at[gg - main_chunks], colv.at[g],
                                 semg)
                pltpu.async_copy(prow_hbm.at[gg - main_chunks], rowv.at[g],
                                 semg)

            @pl.when(g >= 8)
            def _():
                pltpu.make_async_copy(prow_hbm.at[0], colv.at[g - 8],
                                      semg).wait()
                pltpu.make_async_copy(prow_hbm.at[0], rowv.at[g - 8],
                                      semg).wait()
            return carry

        lax.fori_loop(0, c_per_tile, stage_idx, 0)
        for j in range(c_per_tile - 8, c_per_tile):
            pltpu.make_async_copy(prow_hbm.at[0], colv.at[j], semg).wait()
            pltpu.make_async_copy(prow_hbm.at[0], rowv.at[j], semg).wait()
        zero_rows(Q, zb)
        zero_rows(S16, zb16)
        plsc.subcore_barrier()

        half_c = c_per_tile // 2
        stream_phase(r16_hbm, S16, g16, 2, 1, c * half_c, half_c)

        @pl.when(c == 0)
        def _():
            stream_phase(y0_hbm, Q, gbuf, 3, 2, 0, c_per_tile)

        @pl.when(c != 0)
        def _():
            stream_phase(y1_hbm, Q, gbuf, 3, 2, 0, c_per_tile)
        plsc.subcore_barrier()

        flush_rows(t1_hbm)
        pltpu.sync_copy(S16.at[pl.ds(base, seg)],
                        s2_hbm.at[c].at[pl.ds(base, seg)])
        zero_rows(Q, zb)
        plsc.subcore_barrier()

        stream_phase(t1_hbm.at[c], Q, gbuf, 3, 2, 0, c_per_tile)
        plsc.subcore_barrier()
        flush_rows(t2_hbm)
        zero_rows(Q, zb)
        plsc.subcore_barrier()

        stream_phase(t2_hbm.at[c], Q, gbuf, 3, 2, 0, c_per_tile)
        plsc.subcore_barrier()
        flush_rows(a2_hbm)

    return sc_spmm


def kernel(x, edge_index, W, a1, a2):
    del a1
    n, d_in = x.shape
    d_out = W.shape[1]
    h = d_out // 2
    e = edge_index.shape[1]
    f32 = jnp.float32

    n_pad = -(-n // (NS * 8)) * (NS * 8)
    if n_pad == n:
        n_pad += NS * 8
    ept = -(-e // NS)
    ept = -(-ept // (8 * K)) * (8 * K)
    c_per_tile = ept // K
    e_pad = ept * NS - e

    pr = (n + (np.arange(e_pad, dtype=np.int32) % (n_pad - n))).reshape(
        e_pad // K, K)
    pc = ((np.arange(e_pad, dtype=np.int32) * 131) % n).reshape(
        e_pad // K, K)
    pad_row = jnp.asarray(pr, dtype=jnp.int32)
    pad_col = jnp.asarray(pc, dtype=jnp.int32)

    bn = 1000
    grid = n // bn
    y0, y1, r16 = pl.pallas_call(
        _tc_gate_kernel,
        grid=(grid,),
        in_specs=[
            pl.BlockSpec((bn, d_in), lambda i: (i, 0)),
            pl.BlockSpec((d_in, d_out), lambda i: (0, 0)),
            pl.BlockSpec((1, d_out), lambda i: (0, 0)),
        ],
        out_specs=[
            pl.BlockSpec((bn, h), lambda i: (i, 0)),
            pl.BlockSpec((bn, h), lambda i: (i, 0)),
            pl.BlockSpec((bn, LANES), lambda i: (i, 0)),
        ],
        out_shape=[
            jax.ShapeDtypeStruct((n_pad, h), f32),
            jax.ShapeDtypeStruct((n_pad, h), f32),
            jax.ShapeDtypeStruct((n_pad, LANES), f32),
        ],
    )(x, W, a2.reshape(1, d_out))
    sc_spmm = _make_sc_spmm(n_pad, h, c_per_tile, e)
    a2h, s2, _, _ = sc_spmm(y0, y1, r16, edge_index, pad_col, pad_row)

    out = pl.pallas_call(
        _tc_final_kernel,
        grid=(grid,),
        in_specs=[
            pl.BlockSpec((1, bn, h), lambda i: (0, i, 0)),
            pl.BlockSpec((1, bn, h), lambda i: (1, i, 0)),
            pl.BlockSpec((1, bn, LANES), lambda i: (0, i, 0)),
            pl.BlockSpec((1, bn, LANES), lambda i: (1, i, 0)),
        ],
        out_specs=pl.BlockSpec((bn, d_out), lambda i: (i, 0)),
        out_shape=jax.ShapeDtypeStruct((n, d_out), f32),
    )(a2h, a2h, s2, s2)
    return out


# speedup vs baseline: 15.1995x; 1.0189x over previous
import functools

import jax
import jax.numpy as jnp
import numpy as np
from jax import lax
from jax.experimental import pallas as pl
from jax.experimental.pallas import tpu as pltpu
from jax.experimental.pallas import tpu_sc as plsc

NC = 2
NS = 16
LANES = 16
K = 128
ZR = 64


def _tc_gate_kernel(x_ref, w_ref, a2t_ref, y0_ref, y1_ref, r16_ref):
    sup = jnp.dot(x_ref[...], w_ref[...], preferred_element_type=jnp.float32)
    z = jnp.sum(sup * a2t_ref[...], axis=1, keepdims=True)
    z = jnp.where(z >= 0, z, 0.1 * z)
    r = jax.nn.sigmoid(z)
    y = sup * r
    h = y.shape[1] // 2
    y0_ref[...] = y[:, :h]
    y1_ref[...] = y[:, h:]
    r16_ref[...] = jnp.broadcast_to(r, (r.shape[0], LANES))


def _tc_final_kernel(a0_ref, a1_ref, s0_ref, s1_ref, o_ref):
    s = s0_ref[0, :, 0:1] + s1_ref[0, :, 0:1]
    v = jnp.concatenate([a0_ref[0], a1_ref[0]], axis=1) / s
    o_ref[...] = jnp.where(v > 0, v, jnp.exp(jnp.minimum(v, 0.0)) - 1.0)


def _make_sc_spmm(n_pad, h, c_per_tile, e_total):
    seg = n_pad // NS
    mesh = plsc.VectorSubcoreMesh(
        core_axis_name="c", subcore_axis_name="s",
        num_cores=NC, num_subcores=NS)
    f32 = jnp.float32

    @functools.partial(
        pl.kernel,
        out_type=[
            jax.ShapeDtypeStruct((NC, n_pad, h), f32),
            jax.ShapeDtypeStruct((NC, n_pad, LANES), f32),
            jax.ShapeDtypeStruct((NC, n_pad, h), f32),
            jax.ShapeDtypeStruct((NC, n_pad, h), f32),
        ],
        mesh=mesh,
        scratch_types=[
            pltpu.VMEM_SHARED((n_pad, h), f32),
            pltpu.VMEM_SHARED((n_pad, LANES), f32),
            pltpu.VMEM((c_per_tile, K), jnp.int32),
            pltpu.VMEM((c_per_tile, K), jnp.int32),
            pltpu.VMEM((3, K, h), f32),
            pltpu.VMEM((2, K, LANES), f32),
            pltpu.VMEM((ZR, h), f32),
            pltpu.VMEM((ZR, LANES), f32),
            pltpu.SemaphoreType.DMA,
            pltpu.SemaphoreType.DMA,
        ],
        compiler_params=pltpu.CompilerParams(use_tc_tiling_on_sc=False),
    )
    def sc_spmm(y0_hbm, y1_hbm, r16_hbm, eim_hbm, tail_hbm,
                a2_hbm, s2_hbm, t1_hbm, t2_hbm,
                Q, S16, colv, rowv, gbuf, g16, zb, zb16,
                semg, sems):
        c = lax.axis_index("c")
        s = lax.axis_index("s")
        base = s * seg

        def stream_phase(src, dst, bufs, depth, sd, lo, n_chunks):
            def fire_g(j):
                pltpu.async_copy(src.at[colv.at[lo + j]],
                                 bufs.at[j % depth], semg)

            def drain_g(j):
                pltpu.make_async_copy(src.at[colv.at[lo + j]],
                                      bufs.at[j % depth], semg).wait()

            def fire_s(j):
                pltpu.async_copy(bufs.at[j % depth],
                                 dst.at[rowv.at[lo + j]], sems, add=True)

            def drain_s(j):
                pltpu.make_async_copy(bufs.at[j % depth],
                                      dst.at[rowv.at[lo + j]], sems).wait()

            for j in range(depth - sd):
                fire_g(j)

            def body(g, carry):
                @pl.when(g >= sd)
                def _():
                    drain_s(g - sd)

                @pl.when(g + (depth - sd) < n_chunks)
                def _():
                    fire_g(g + (depth - sd))

                drain_g(g)
                fire_s(g)
                return carry

            lax.fori_loop(0, n_chunks, body, 0)
            for j in range(n_chunks - sd, n_chunks):
                drain_s(j)

        def zero_rows(dst, zsrc):
            nfull = seg // ZR
            rem = seg - nfull * ZR
            for t in range(nfull):
                pltpu.sync_copy(zsrc, dst.at[pl.ds(base + t * ZR, ZR)])
            if rem:
                pltpu.sync_copy(zsrc.at[pl.ds(0, rem)],
                                dst.at[pl.ds(base + nfull * ZR, rem)])

        def flush_rows(dst2):
            pltpu.sync_copy(Q.at[pl.ds(base, seg)],
                            dst2.at[c].at[pl.ds(base, seg)])

        zv = jnp.zeros((LANES,), f32)
        for i in range(ZR):
            for q in range(h // LANES):
                zb[i, pl.ds(q * LANES, LANES)] = zv
            zb16[i, pl.ds(0, LANES)] = zv

        @pl.when(s < NS - 1)
        def _():
            pltpu.sync_copy(eim_hbm.at[0].at[pl.ds(s * c_per_tile,
                                                   c_per_tile)], colv)
            pltpu.sync_copy(eim_hbm.at[1].at[pl.ds(s * c_per_tile,
                                                   c_per_tile)], rowv)

        @pl.when(s == NS - 1)
        def _():
            pltpu.sync_copy(tail_hbm.at[0], colv)
            pltpu.sync_copy(tail_hbm.at[1], rowv)

        zero_rows(Q, zb)
        zero_rows(S16, zb16)
        plsc.subcore_barrier()

        half_c = c_per_tile // 2
        stream_phase(r16_hbm, S16, g16, 2, 1, c * half_c, half_c)

        @pl.when(c == 0)
        def _():
            stream_phase(y0_hbm, Q, gbuf, 3, 2, 0, c_per_tile)

        @pl.when(c != 0)
        def _():
            stream_phase(y1_hbm, Q, gbuf, 3, 2, 0, c_per_tile)
        plsc.subcore_barrier()

        flush_rows(t1_hbm)
        pltpu.sync_copy(S16.at[pl.ds(base, seg)],
                        s2_hbm.at[c].at[pl.ds(base, seg)])
        zero_rows(Q, zb)
        plsc.subcore_barrier()

        stream_phase(t1_hbm.at[c], Q, gbuf, 3, 2, 0, c_per_tile)
        plsc.subcore_barrier()
        flush_rows(t2_hbm)
        zero_rows(Q, zb)
        plsc.subcore_barrier()

        stream_phase(t2_hbm.at[c], Q, gbuf, 3, 2, 0, c_per_tile)
        plsc.subcore_barrier()
        flush_rows(a2_hbm)

    return sc_spmm


def kernel(x, edge_index, W, a1, a2):
    del a1
    n, d_in = x.shape
    d_out = W.shape[1]
    h = d_out // 2
    e = edge_index.shape[1]
    f32 = jnp.float32

    n_pad = -(-n // (NS * 8)) * (NS * 8)
    if n_pad == n:
        n_pad += NS * 8
    ept = -(-e // NS)
    ept = -(-ept // (8 * K)) * (8 * K)
    c_per_tile = ept // K
    e_pad = ept * NS - e

    pr = n + (np.arange(e_pad, dtype=np.int32) % (n_pad - n))
    pc = (np.arange(e_pad, dtype=np.int32) * 131) % n
    pad_pairs = jnp.asarray(np.stack([pc, pr]), dtype=jnp.int32)
    eim = edge_index.reshape(2, e // K, K)
    ept = c_per_tile * K
    tail = jnp.concatenate(
        [edge_index[:, (NS - 1) * ept:], pad_pairs], axis=1
    ).reshape(2, c_per_tile, K)

    bn = 1000
    grid = n // bn
    y0, y1, r16 = pl.pallas_call(
        _tc_gate_kernel,
        grid=(grid,),
        in_specs=[
            pl.BlockSpec((bn, d_in), lambda i: (i, 0)),
            pl.BlockSpec((d_in, d_out), lambda i: (0, 0)),
            pl.BlockSpec((1, d_out), lambda i: (0, 0)),
        ],
        out_specs=[
            pl.BlockSpec((bn, h), lambda i: (i, 0)),
            pl.BlockSpec((bn, h), lambda i: (i, 0)),
            pl.BlockSpec((bn, LANES), lambda i: (i, 0)),
        ],
        out_shape=[
            jax.ShapeDtypeStruct((n_pad, h), f32),
            jax.ShapeDtypeStruct((n_pad, h), f32),
            jax.ShapeDtypeStruct((n_pad, LANES), f32),
        ],
    )(x, W, a2.reshape(1, d_out))
    sc_spmm = _make_sc_spmm(n_pad, h, c_per_tile, e)
    a2h, s2, _, _ = sc_spmm(y0, y1, r16, eim, tail)

    out = pl.pallas_call(
        _tc_final_kernel,
        grid=(grid,),
        in_specs=[
            pl.BlockSpec((1, bn, h), lambda i: (0, i, 0)),
            pl.BlockSpec((1, bn, h), lambda i: (1, i, 0)),
            pl.BlockSpec((1, bn, LANES), lambda i: (0, i, 0)),
            pl.BlockSpec((1, bn, LANES), lambda i: (1, i, 0)),
        ],
        out_specs=pl.BlockSpec((bn, d_out), lambda i: (i, 0)),
        out_shape=jax.ShapeDtypeStruct((n, d_out), f32),
    )(a2h, a2h, s2, s2)
    return out
